# baseline (speedup 1.0000x reference)
import functools

import jax
import jax.numpy as jnp
from jax import lax
from jax.experimental import pallas as pl
from jax.experimental.pallas import tpu as pltpu
from jax.experimental.pallas import tpu_sc as plsc

VOCAB = 1000000
EMBED = 64
BATCH = 16384

L = 16
NC = 2
NS = 16
NW = NC * NS
BPW = BATCH // NW
NG = BPW // L

_LN2 = 0.6931471805599453
_SQRT2 = 1.4142135623730951


def _vlog(x):
    bits = lax.bitcast_convert_type(x, jnp.int32)
    e = lax.shift_right_arithmetic(bits, 23) - 127
    m = lax.bitcast_convert_type(
        (bits & 0x7FFFFF) | 0x3F800000, jnp.float32)
    big = m >= _SQRT2
    m = jnp.where(big, m * 0.5, m)
    e = (e + jnp.where(big, 1, 0)).astype(jnp.float32)
    t = (m - 1.0) / (m + 1.0)
    t2 = t * t
    p = 1.0 + t2 * (1 / 3 + t2 * (1 / 5 + t2 * (1 / 7 + t2 * (1 / 9))))
    lnm = 2.0 * t * p
    return jnp.where(x <= 0.0, -jnp.inf, e * _LN2 + lnm)


def _tec_body(i_hbm, k_hbm, x_hbm, w_hbm, Wt_hbm, but_hbm, bvt_hbm, out_hbm,
              idx_i, idx_k, wi_t, wk_t, bu_v, bv_v, xv, wv, accv, s1, s2):
    wid = lax.axis_index("s") * NC + lax.axis_index("c")
    base = pl.multiple_of(wid * BPW, BPW)

    pltpu.sync_copy(i_hbm.at[pl.ds(base, BPW)], idx_i)
    pltpu.sync_copy(k_hbm.at[pl.ds(base, BPW)], idx_k)
    copies = []
    for d in range(EMBED):
        copies.append(
            pltpu.async_copy(Wt_hbm.at[d].at[idx_i], wi_t.at[d], s1))
        copies.append(
            pltpu.async_copy(Wt_hbm.at[d].at[idx_k], wk_t.at[d], s1))
    c3 = pltpu.async_copy(but_hbm.at[0].at[idx_i], bu_v, s2)
    c4 = pltpu.async_copy(bvt_hbm.at[0].at[idx_k], bv_v, s2)
    pltpu.sync_copy(x_hbm.at[pl.ds(base, BPW)], xv)
    pltpu.sync_copy(w_hbm.at[pl.ds(base, BPW)], wv)
    for c in copies:
        c.wait()
    c3.wait()
    c4.wait()

    def group(g, acc):
        gb = pl.multiple_of(g * L, L)
        sim = jnp.zeros((L,), jnp.float32)
        for d in range(EMBED):
            sim = sim + wi_t[d, pl.ds(gb, L)] * wk_t[d, pl.ds(gb, L)]
        bu16 = bu_v[pl.ds(gb, L)]
        bv16 = bv_v[pl.ds(gb, L)]
        x16 = xv[pl.ds(gb, L)]
        w16 = wv[pl.ds(gb, L)]
        r = sim + bu16 + bv16 - _vlog(x16)
        return acc + r * r * w16 * 0.5

    acc = lax.fori_loop(0, NG, group, jnp.zeros((L,), jnp.float32))
    accv[...] = acc
    pltpu.sync_copy(accv, out_hbm.at[wid])


@jax.jit
def _glove_sc(i, k, x_ik, w, Wt, but, bvt):
    mesh = plsc.VectorSubcoreMesh(core_axis_name="c", subcore_axis_name="s")
    f = functools.partial(
        pl.kernel,
        mesh=mesh,
        compiler_params=pltpu.CompilerParams(
            needs_layout_passes=False, use_tc_tiling_on_sc=False),
        out_type=jax.ShapeDtypeStruct((NW, L), jnp.float32),
        scratch_types=[
            pltpu.VMEM((BPW,), jnp.int32),
            pltpu.VMEM((BPW,), jnp.int32),
            pltpu.VMEM((EMBED, BPW), jnp.float32),
            pltpu.VMEM((EMBED, BPW), jnp.float32),
            pltpu.VMEM((BPW,), jnp.float32),
            pltpu.VMEM((BPW,), jnp.float32),
            pltpu.VMEM((BPW,), jnp.float32),
            pltpu.VMEM((BPW,), jnp.float32),
            pltpu.VMEM((L,), jnp.float32),
            pltpu.SemaphoreType.DMA,
            pltpu.SemaphoreType.DMA,
        ],
    )(_tec_body)
    return f(i, k, x_ik, w, Wt, but, bvt)


def kernel(i, k, x_ik, w, W, B_v, B_u):
    partials = _glove_sc(i, k, x_ik, w, W.T, B_u.T, B_v.T)
    return jnp.sum(partials)


# speedup vs baseline: 8.1293x; 8.1293x over previous
import functools

import jax
import jax.numpy as jnp
from jax import lax
from jax.experimental import pallas as pl
from jax.experimental.pallas import tpu as pltpu
from jax.experimental.pallas import tpu_sc as plsc

VOCAB = 1000000
EMBED = 64
PADW = 128
BATCH = 16384

L = 16
NC = 2
NS = 16
NW = NC * NS
BPW = BATCH // NW
SUB = 128
NSUB = BPW // SUB
NGS = SUB // L

_LN2 = 0.6931471805599453
_SQRT2 = 1.4142135623730951


def _vlog(x):
    bits = lax.bitcast_convert_type(x, jnp.int32)
    e = lax.shift_right_arithmetic(bits, 23) - 127
    m = lax.bitcast_convert_type(
        (bits & 0x7FFFFF) | 0x3F800000, jnp.float32)
    big = m >= _SQRT2
    m = jnp.where(big, m * 0.5, m)
    e = (e + jnp.where(big, 1, 0)).astype(jnp.float32)
    t = (m - 1.0) / (m + 1.0)
    t2 = t * t
    p = 1.0 + t2 * (1 / 3 + t2 * (1 / 5 + t2 * (1 / 7 + t2 * (1 / 9))))
    lnm = 2.0 * t * p
    return jnp.where(x <= 0.0, -jnp.inf, e * _LN2 + lnm)


def _tec_body(i_hbm, k_hbm, x_hbm, w_hbm, Wp_hbm, out_hbm,
              idx_i, idx_k, wi_b, wk_b, xv, wv, accv, sem0, sem1):
    wid = lax.axis_index("s") * NC + lax.axis_index("c")
    base = pl.multiple_of(wid * BPW, BPW)

    pltpu.sync_copy(i_hbm.at[pl.ds(base, BPW)], idx_i)
    pltpu.sync_copy(k_hbm.at[pl.ds(base, BPW)], idx_k)
    pltpu.sync_copy(x_hbm.at[pl.ds(base, BPW)], xv)
    pltpu.sync_copy(w_hbm.at[pl.ds(base, BPW)], wv)

    def fire(s):
        buf = s % 2
        sem = sem0 if buf == 0 else sem1
        sl = pl.ds(s * SUB, SUB)
        return (
            pltpu.async_copy(Wp_hbm.at[idx_i.at[sl]], wi_b.at[buf], sem),
            pltpu.async_copy(Wp_hbm.at[idx_k.at[sl]], wk_b.at[buf], sem),
        )

    lane = lax.iota(jnp.int32, L)
    inflight = fire(0)
    acc = jnp.zeros((L,), jnp.float32)
    for s in range(NSUB):
        buf = s % 2
        for c in inflight:
            c.wait()
        if s + 1 < NSUB:
            inflight = fire(s + 1)

        def group(g, a, s=s, buf=buf):
            sims = jnp.zeros((L,), jnp.float32)
            for b in range(L):
                p = g * L + b
                pr = None
                for c in range(EMBED // L):
                    ai = wi_b[buf, p, pl.ds(c * L, L)]
                    ak = wk_b[buf, p, pl.ds(c * L, L)]
                    pr = ai * ak if pr is None else pr + ai * ak
                dot = jnp.sum(pr)
                sims = jnp.where(lane == b, dot, sims)
            gb = pl.multiple_of(s * SUB + g * L, L)
            x16 = xv[pl.ds(gb, L)]
            w16 = wv[pl.ds(gb, L)]
            r = sims - _vlog(x16)
            return a + r * r * w16 * 0.5

        acc = lax.fori_loop(0, NGS, group, acc)

    accv[...] = acc
    pltpu.sync_copy(accv, out_hbm.at[wid])


@jax.jit
def _glove_sc(i, k, x_ik, w, Wp):
    mesh = plsc.VectorSubcoreMesh(core_axis_name="c", subcore_axis_name="s")
    f = functools.partial(
        pl.kernel,
        mesh=mesh,
        compiler_params=pltpu.CompilerParams(needs_layout_passes=False),
        out_type=jax.ShapeDtypeStruct((NW, L), jnp.float32),
        scratch_types=[
            pltpu.VMEM((BPW,), jnp.int32),
            pltpu.VMEM((BPW,), jnp.int32),
            pltpu.VMEM((2, SUB, PADW), jnp.float32),
            pltpu.VMEM((2, SUB, PADW), jnp.float32),
            pltpu.VMEM((BPW,), jnp.float32),
            pltpu.VMEM((BPW,), jnp.float32),
            pltpu.VMEM((L,), jnp.float32),
            pltpu.SemaphoreType.DMA,
            pltpu.SemaphoreType.DMA,
        ],
    )(_tec_body)
    return f(i, k, x_ik, w, Wp)


def kernel(i, k, x_ik, w, W, B_v, B_u):
    Wp = jnp.pad(W, ((0, 0), (0, PADW - EMBED)))
    bu_g = jnp.take(B_u, i, axis=0).reshape(BATCH)
    bv_g = jnp.take(B_v, k, axis=0).reshape(BATCH)
    xb = x_ik * jnp.exp(-bu_g - bv_g)
    partials = _glove_sc(i, k, xb, w, Wp)
    return jnp.sum(partials)


# speedup vs baseline: 9.5001x; 1.1686x over previous
import functools

import jax
import jax.numpy as jnp
from jax import lax
from jax.experimental import pallas as pl
from jax.experimental.pallas import tpu as pltpu
from jax.experimental.pallas import tpu_sc as plsc

VOCAB = 1000000
EMBED = 64
PADW = 128
BATCH = 16384

L = 16
NC = 2
NS = 16
NW = NC * NS
BPW = BATCH // NW
SUB = 128
NSUB = BPW // SUB
NGS = SUB // L

_LN2 = 0.6931471805599453
_SQRT2 = 1.4142135623730951


def _vlog(x):
    bits = lax.bitcast_convert_type(x, jnp.int32)
    e = lax.shift_right_arithmetic(bits, 23) - 127
    m = lax.bitcast_convert_type(
        (bits & 0x7FFFFF) | 0x3F800000, jnp.float32)
    big = m >= _SQRT2
    m = jnp.where(big, m * 0.5, m)
    e = (e + jnp.where(big, 1, 0)).astype(jnp.float32)
    t = (m - 1.0) / (m + 1.0)
    t2 = t * t
    p = 1.0 + t2 * (1 / 3 + t2 * (1 / 5 + t2 * (1 / 7 + t2 * (1 / 9))))
    lnm = 2.0 * t * p
    return jnp.where(x <= 0.0, -jnp.inf, e * _LN2 + lnm)


def _tec_body(i_hbm, k_hbm, x_hbm, w_hbm, Wp_hbm, bu_hbm, bv_hbm, out_hbm,
              idx_i, idx_k, wi_b, wk_b, bu_v, bv_v, xv, wv, accv,
              sem0, sem1, sem2):
    wid = lax.axis_index("s") * NC + lax.axis_index("c")
    base = pl.multiple_of(wid * BPW, BPW)

    pltpu.sync_copy(i_hbm.at[pl.ds(base, BPW)], idx_i)
    pltpu.sync_copy(k_hbm.at[pl.ds(base, BPW)], idx_k)
    cb1 = pltpu.async_copy(bu_hbm.at[idx_i], bu_v, sem2)
    cb2 = pltpu.async_copy(bv_hbm.at[idx_k], bv_v, sem2)
    pltpu.sync_copy(x_hbm.at[pl.ds(base, BPW)], xv)
    pltpu.sync_copy(w_hbm.at[pl.ds(base, BPW)], wv)

    def fire(s):
        buf = s % 2
        sem = sem0 if buf == 0 else sem1
        sl = pl.ds(s * SUB, SUB)
        return (
            pltpu.async_copy(Wp_hbm.at[idx_i.at[sl]], wi_b.at[buf], sem),
            pltpu.async_copy(Wp_hbm.at[idx_k.at[sl]], wk_b.at[buf], sem),
        )

    lane = lax.iota(jnp.int32, L)
    inflight = fire(0)
    cb1.wait()
    cb2.wait()
    acc = jnp.zeros((L,), jnp.float32)
    for s in range(NSUB):
        buf = s % 2
        for c in inflight:
            c.wait()
        if s + 1 < NSUB:
            inflight = fire(s + 1)

        def group(g, a, s=s, buf=buf):
            sims = jnp.zeros((L,), jnp.float32)
            for b in range(L):
                p = g * L + b
                pr = None
                for c in range(EMBED // L):
                    ai = wi_b[buf, p, pl.ds(c * L, L)]
                    ak = wk_b[buf, p, pl.ds(c * L, L)]
                    pr = ai * ak if pr is None else pr + ai * ak
                dot = jnp.sum(pr)
                sims = jnp.where(lane == b, dot, sims)
            gb = pl.multiple_of(s * SUB + g * L, L)
            x16 = xv[pl.ds(gb, L)]
            w16 = wv[pl.ds(gb, L)]
            bu16 = bu_v[pl.ds(gb, L)]
            bv16 = bv_v[pl.ds(gb, L)]
            r = sims + bu16 + bv16 - _vlog(x16)
            return a + r * r * w16 * 0.5

        acc = lax.fori_loop(0, NGS, group, acc)

    accv[...] = acc
    pltpu.sync_copy(accv, out_hbm.at[wid])


@jax.jit
def _glove_sc(i, k, x_ik, w, Wp, bu, bv):
    mesh = plsc.VectorSubcoreMesh(core_axis_name="c", subcore_axis_name="s")
    f = functools.partial(
        pl.kernel,
        mesh=mesh,
        compiler_params=pltpu.CompilerParams(needs_layout_passes=False),
        out_type=jax.ShapeDtypeStruct((NW, L), jnp.float32),
        scratch_types=[
            pltpu.VMEM((BPW,), jnp.int32),
            pltpu.VMEM((BPW,), jnp.int32),
            pltpu.VMEM((2, SUB, PADW), jnp.float32),
            pltpu.VMEM((2, SUB, PADW), jnp.float32),
            pltpu.VMEM((BPW,), jnp.float32),
            pltpu.VMEM((BPW,), jnp.float32),
            pltpu.VMEM((BPW,), jnp.float32),
            pltpu.VMEM((BPW,), jnp.float32),
            pltpu.VMEM((L,), jnp.float32),
            pltpu.SemaphoreType.DMA,
            pltpu.SemaphoreType.DMA,
            pltpu.SemaphoreType.DMA,
        ],
    )(_tec_body)
    return f(i, k, x_ik, w, Wp, bu, bv)


def kernel(i, k, x_ik, w, W, B_v, B_u):
    Wp = jnp.pad(W, ((0, 0), (0, PADW - EMBED)))
    partials = _glove_sc(i, k, x_ik, w, Wp,
                         B_u.reshape(VOCAB), B_v.reshape(VOCAB))
    return jnp.sum(partials)


# speedup vs baseline: 11.5377x; 1.2145x over previous
import functools

import jax
import jax.numpy as jnp
from jax import lax
from jax.experimental import pallas as pl
from jax.experimental.pallas import tpu as pltpu
from jax.experimental.pallas import tpu_sc as plsc

VOCAB = 1000000
EMBED = 64
BATCH = 16384

L = 16
NC = 2
NS = 16
NW = NC * NS
BPW = BATCH // NW
SUB = 128
NSUB = BPW // SUB

_LN2 = 0.6931471805599453
_SQRT2 = 1.4142135623730951


def _vlog(x):
    bits = lax.bitcast_convert_type(x, jnp.int32)
    e = lax.shift_right_arithmetic(bits, 23) - 127
    m = lax.bitcast_convert_type(
        (bits & 0x7FFFFF) | 0x3F800000, jnp.float32)
    big = m >= _SQRT2
    m = jnp.where(big, m * 0.5, m)
    e = (e + jnp.where(big, 1, 0)).astype(jnp.float32)
    t = (m - 1.0) / (m + 1.0)
    t2 = t * t
    p = 1.0 + t2 * (1 / 3 + t2 * (1 / 5 + t2 * (1 / 7 + t2 * (1 / 9))))
    lnm = 2.0 * t * p
    return jnp.where(x <= 0.0, -jnp.inf, e * _LN2 + lnm)


def _tec_body(i_hbm, k_hbm, x_hbm, w_hbm, W_hbm, out_hbm,
              idx_i, idx_k, wi_b, wk_b, xv, wv, accv, sem0, sem1):
    wid = lax.axis_index("s") * NC + lax.axis_index("c")
    base = pl.multiple_of(wid * BPW, BPW)

    pltpu.sync_copy(i_hbm.at[pl.ds(base, BPW)], idx_i)
    pltpu.sync_copy(k_hbm.at[pl.ds(base, BPW)], idx_k)
    pltpu.sync_copy(x_hbm.at[pl.ds(base, BPW)], xv)
    pltpu.sync_copy(w_hbm.at[pl.ds(base, BPW)], wv)

    def fire(s, buf, sem):
        soff = s * SUB

        def one(g, _):
            gv = pl.multiple_of(g * L, L)
            iv = idx_i[pl.ds(soff + gv, L)]
            kv = idx_k[pl.ds(soff + gv, L)]
            for b in range(L):
                p = gv + b
                pltpu.async_copy(W_hbm.at[pl.ds(iv[b], 1)],
                                 wi_b.at[buf].at[pl.ds(p, 1)], sem)
                pltpu.async_copy(W_hbm.at[pl.ds(kv[b], 1)],
                                 wk_b.at[buf].at[pl.ds(p, 1)], sem)
            return 0

        lax.fori_loop(0, SUB // L, one, 0)

    def drain(buf, sem):
        pltpu.make_async_copy(W_hbm.at[pl.ds(0, SUB)], wi_b.at[buf],
                              sem).wait()
        pltpu.make_async_copy(W_hbm.at[pl.ds(0, SUB)], wk_b.at[buf],
                              sem).wait()

    lane = lax.iota(jnp.int32, L)
    fire(0, 0, sem0)
    acc = jnp.zeros((L,), jnp.float32)
    for s in range(NSUB):
        buf = s % 2
        drain(buf, sem0 if buf == 0 else sem1)
        if s + 1 < NSUB:
            fire(s + 1, 1 - buf, sem1 if buf == 0 else sem0)

        def group(g, a, s=s, buf=buf):
            sims = jnp.zeros((L,), jnp.float32)
            for b in range(L):
                p = g * L + b
                pr = None
                for c in range(EMBED // L):
                    ai = wi_b[buf, p, pl.ds(c * L, L)]
                    ak = wk_b[buf, p, pl.ds(c * L, L)]
                    pr = ai * ak if pr is None else pr + ai * ak
                sims = jnp.where(lane == b, jnp.sum(pr), sims)
            gb = pl.multiple_of(s * SUB + g * L, L)
            x16 = xv[pl.ds(gb, L)]
            w16 = wv[pl.ds(gb, L)]
            r = sims - _vlog(x16)
            return a + r * r * w16 * 0.5

        acc = lax.fori_loop(0, SUB // L, group, acc)

    accv[...] = acc
    pltpu.sync_copy(accv, out_hbm.at[wid])


@jax.jit
def _glove_sc(i, k, x_ik, w, W):
    mesh = plsc.VectorSubcoreMesh(core_axis_name="c", subcore_axis_name="s")
    f = functools.partial(
        pl.kernel,
        mesh=mesh,
        compiler_params=pltpu.CompilerParams(needs_layout_passes=False),
        out_type=jax.ShapeDtypeStruct((NW, L), jnp.float32),
        scratch_types=[
            pltpu.VMEM((BPW,), jnp.int32),
            pltpu.VMEM((BPW,), jnp.int32),
            pltpu.VMEM((2, SUB, EMBED), jnp.float32),
            pltpu.VMEM((2, SUB, EMBED), jnp.float32),
            pltpu.VMEM((BPW,), jnp.float32),
            pltpu.VMEM((BPW,), jnp.float32),
            pltpu.VMEM((L,), jnp.float32),
            pltpu.SemaphoreType.DMA,
            pltpu.SemaphoreType.DMA,
        ],
    )(_tec_body)
    return f(i, k, x_ik, w, W)


def kernel(i, k, x_ik, w, W, B_v, B_u):
    bu_g = jnp.take(B_u, i, axis=0).reshape(BATCH)
    bv_g = jnp.take(B_v, k, axis=0).reshape(BATCH)
    xb = x_ik * jnp.exp(-bu_g - bv_g)
    partials = _glove_sc(i, k, xb, w, W)
    return jnp.sum(partials)
